# baseline (speedup 1.0000x reference)
import jax
import jax.numpy as jnp
from jax import lax
from jax.experimental import pallas as pl
from jax.experimental.pallas import tpu as pltpu

N_RAYS = 32768
N_SURF = 16
NBOUNCES = 100
EPS = 1e-4
LANES = 128
ROWS = N_RAYS // LANES
BLK = 16
GRID = ROWS // BLK


def _body(ctr_ref, rad_ref, refl_ref,
          px_ref, py_ref, pz_ref, dx_ref, dy_ref, dz_ref, it_ref,
          opx_ref, opy_ref, opz_ref, odx_ref, ody_ref, odz_ref, oit_ref):
    px = px_ref[...] * 5.0
    py = py_ref[...] * 5.0
    pz = pz_ref[...] * 5.0
    dx = dx_ref[...]
    dy = dy_ref[...]
    dz = dz_ref[...]
    inten = it_ref[...]

    nrm = jnp.sqrt((dx * dx + dz * dz) + dy * dy) + 1e-12
    dx = dx / nrm
    dy = dy / nrm
    dz = dz / nrm

    inf = jnp.float32(jnp.inf)

    surf = []
    for s in range(N_SURF):
        surf.append((ctr_ref[s, 0] * 10.0, ctr_ref[s, 1] * 10.0,
                     ctr_ref[s, 2] * 10.0, rad_ref[s] * 4.0 + 1.0,
                     refl_ref[s]))

    def chain(state, lo, hi):
        px, py, pz, dx, dy, dz = state
        best_t = jnp.full(px.shape, inf, jnp.float32)
        bcx = jnp.zeros(px.shape, jnp.float32)
        bcy = jnp.zeros(px.shape, jnp.float32)
        bcz = jnp.zeros(px.shape, jnp.float32)
        brad = jnp.ones(px.shape, jnp.float32)
        brefl = jnp.zeros(px.shape, jnp.float32)
        for s in range(lo, hi):
            cx, cy, cz, r, rf = surf[s]
            ox = px - cx
            oy = py - cy
            oz = pz - cz
            b = (ox * dx + oy * dy) + oz * dz
            c = ((ox * ox + oy * oy) + oz * oz) - r * r
            disc = b * b - c
            valid = disc > 0.0
            sq = jnp.sqrt(disc)
            t0 = -b - sq
            t1 = -b + sq
            t = jnp.where(valid & (t0 > EPS), t0,
                          jnp.where(valid & (t1 > EPS), t1, inf))
            take = t < best_t
            best_t = jnp.where(take, t, best_t)
            bcx = jnp.where(take, cx, bcx)
            bcy = jnp.where(take, cy, bcy)
            bcz = jnp.where(take, cz, bcz)
            brad = jnp.where(take, r, brad)
            brefl = jnp.where(take, rf, brefl)
        return best_t, bcx, bcy, bcz, brad, brefl

    def bounce(_, state):
        px, py, pz, dx, dy, dz, inten = state
        rays = (px, py, pz, dx, dy, dz)
        half = N_SURF // 2
        ta, axc, ayc, azc, ar, af = chain(rays, 0, half)
        tb, bxc, byc, bzc, br, bf = chain(rays, half, N_SURF)
        tk = tb < ta
        best_t = jnp.where(tk, tb, ta)
        bcx = jnp.where(tk, bxc, axc)
        bcy = jnp.where(tk, byc, ayc)
        bcz = jnp.where(tk, bzc, azc)
        brad = jnp.where(tk, br, ar)
        brefl = jnp.where(tk, bf, af)
        hit = best_t < inf
        active = hit & (inten > 0.0)
        t_safe = jnp.where(hit, best_t, 0.0)
        hx = px + t_safe * dx
        hy = py + t_safe * dy
        hz = pz + t_safe * dz
        nx = (hx - bcx) / brad
        ny = (hy - bcy) / brad
        nz = (hz - bcz) / brad
        dn = (dx * nx + dz * nz) + dy * ny
        k2 = 2.0 * dn
        ndx = dx - k2 * nx
        ndy = dy - k2 * ny
        ndz = dz - k2 * nz
        ni = inten * brefl
        px = jnp.where(active, hx, px)
        py = jnp.where(active, hy, py)
        pz = jnp.where(active, hz, pz)
        dx = jnp.where(active, ndx, dx)
        dy = jnp.where(active, ndy, dy)
        dz = jnp.where(active, ndz, dz)
        inten = jnp.where(active, ni, inten)
        return px, py, pz, dx, dy, dz, inten

    px, py, pz, dx, dy, dz, inten = lax.fori_loop(
        0, NBOUNCES, bounce, (px, py, pz, dx, dy, dz, inten))

    opx_ref[...] = px
    opy_ref[...] = py
    opz_ref[...] = pz
    odx_ref[...] = dx
    ody_ref[...] = dy
    odz_ref[...] = dz
    oit_ref[...] = inten


def kernel(pos, dir, intensity, centers, radii, reflectivity):
    comps = [pos[:, 0], pos[:, 1], pos[:, 2],
             dir[:, 0], dir[:, 1], dir[:, 2], intensity]
    planes = [c.reshape(ROWS, LANES) for c in comps]

    vspec = pl.BlockSpec((BLK, LANES), lambda i: (i, 0))
    sspec = pl.BlockSpec(memory_space=pltpu.MemorySpace.SMEM)
    out = pl.pallas_call(
        _body,
        grid=(GRID,),
        in_specs=[sspec, sspec, sspec] + [vspec] * 7,
        out_specs=[vspec] * 7,
        out_shape=[jax.ShapeDtypeStruct((ROWS, LANES), jnp.float32)] * 7,
    )(centers, radii, reflectivity, *planes)

    flat = [o.reshape(N_RAYS) for o in out]
    return jnp.stack(flat, axis=1)


# speedup vs baseline: 1.0764x; 1.0764x over previous
import jax
import jax.numpy as jnp
from jax import lax
from jax.experimental import pallas as pl
from jax.experimental.pallas import tpu as pltpu

N_RAYS = 32768
N_SURF = 16
NBOUNCES = 100
EPS = 1e-4
LANES = 128
ROWS = N_RAYS // LANES
BLK = 16
GRID = ROWS // BLK


def _body(ctr_ref, rad_ref, refl_ref,
          px_ref, py_ref, pz_ref, dx_ref, dy_ref, dz_ref, it_ref,
          opx_ref, opy_ref, opz_ref, odx_ref, ody_ref, odz_ref, oit_ref):
    px = px_ref[...] * 5.0
    py = py_ref[...] * 5.0
    pz = pz_ref[...] * 5.0
    dx = dx_ref[...]
    dy = dy_ref[...]
    dz = dz_ref[...]
    inten = it_ref[...]

    nrm = jnp.sqrt((dx * dx + dz * dz) + dy * dy) + 1e-12
    dx = dx / nrm
    dy = dy / nrm
    dz = dz / nrm

    inf = jnp.float32(jnp.inf)

    surf = []
    for s in range(N_SURF):
        surf.append((ctr_ref[s, 0] * 10.0, ctr_ref[s, 1] * 10.0,
                     ctr_ref[s, 2] * 10.0, rad_ref[s] * 4.0 + 1.0,
                     refl_ref[s]))

    def chain(state, lo, hi):
        px, py, pz, dx, dy, dz = state
        best_t = jnp.full(px.shape, inf, jnp.float32)
        bcx = jnp.zeros(px.shape, jnp.float32)
        bcy = jnp.zeros(px.shape, jnp.float32)
        bcz = jnp.zeros(px.shape, jnp.float32)
        brad = jnp.ones(px.shape, jnp.float32)
        brefl = jnp.zeros(px.shape, jnp.float32)
        for s in range(lo, hi):
            cx, cy, cz, r, rf = surf[s]
            ox = px - cx
            oy = py - cy
            oz = pz - cz
            b = (ox * dx + oy * dy) + oz * dz
            c = ((ox * ox + oy * oy) + oz * oz) - r * r
            disc = b * b - c
            valid = disc > 0.0
            sq = jnp.sqrt(disc)
            t0 = -b - sq
            t1 = -b + sq
            t = jnp.where(valid & (t0 > EPS), t0,
                          jnp.where(valid & (t1 > EPS), t1, inf))
            take = t < best_t
            best_t = jnp.where(take, t, best_t)
            bcx = jnp.where(take, cx, bcx)
            bcy = jnp.where(take, cy, bcy)
            bcz = jnp.where(take, cz, bcz)
            brad = jnp.where(take, r, brad)
            brefl = jnp.where(take, rf, brefl)
        return best_t, bcx, bcy, bcz, brad, brefl

    def bounce(_, state):
        px, py, pz, dx, dy, dz, inten = state
        rays = (px, py, pz, dx, dy, dz)
        half = N_SURF // 2
        ta, axc, ayc, azc, ar, af = chain(rays, 0, half)
        tb, bxc, byc, bzc, br, bf = chain(rays, half, N_SURF)
        tk = tb < ta
        best_t = jnp.where(tk, tb, ta)
        bcx = jnp.where(tk, bxc, axc)
        bcy = jnp.where(tk, byc, ayc)
        bcz = jnp.where(tk, bzc, azc)
        brad = jnp.where(tk, br, ar)
        brefl = jnp.where(tk, bf, af)
        hit = best_t < inf
        active = hit & (inten > 0.0)
        t_safe = jnp.where(hit, best_t, 0.0)
        hx = px + t_safe * dx
        hy = py + t_safe * dy
        hz = pz + t_safe * dz
        nx = (hx - bcx) / brad
        ny = (hy - bcy) / brad
        nz = (hz - bcz) / brad
        dn = (dx * nx + dz * nz) + dy * ny
        k2 = 2.0 * dn
        ndx = dx - k2 * nx
        ndy = dy - k2 * ny
        ndz = dz - k2 * nz
        ni = inten * brefl
        px = jnp.where(active, hx, px)
        py = jnp.where(active, hy, py)
        pz = jnp.where(active, hz, pz)
        dx = jnp.where(active, ndx, dx)
        dy = jnp.where(active, ndy, dy)
        dz = jnp.where(active, ndz, dz)
        inten = jnp.where(active, ni, inten)
        return px, py, pz, dx, dy, dz, inten

    UNROLL = 4

    def bounce4(i, state):
        for _ in range(UNROLL):
            state = bounce(i, state)
        return state

    px, py, pz, dx, dy, dz, inten = lax.fori_loop(
        0, NBOUNCES // UNROLL, bounce4, (px, py, pz, dx, dy, dz, inten))

    opx_ref[...] = px
    opy_ref[...] = py
    opz_ref[...] = pz
    odx_ref[...] = dx
    ody_ref[...] = dy
    odz_ref[...] = dz
    oit_ref[...] = inten


def kernel(pos, dir, intensity, centers, radii, reflectivity):
    comps = [pos[:, 0], pos[:, 1], pos[:, 2],
             dir[:, 0], dir[:, 1], dir[:, 2], intensity]
    planes = [c.reshape(ROWS, LANES) for c in comps]

    vspec = pl.BlockSpec((BLK, LANES), lambda i: (i, 0))
    sspec = pl.BlockSpec(memory_space=pltpu.MemorySpace.SMEM)
    out = pl.pallas_call(
        _body,
        grid=(GRID,),
        in_specs=[sspec, sspec, sspec] + [vspec] * 7,
        out_specs=[vspec] * 7,
        out_shape=[jax.ShapeDtypeStruct((ROWS, LANES), jnp.float32)] * 7,
    )(centers, radii, reflectivity, *planes)

    flat = [o.reshape(N_RAYS) for o in out]
    return jnp.stack(flat, axis=1)


# speedup vs baseline: 1.0981x; 1.0201x over previous
import jax
import jax.numpy as jnp
from jax import lax
from jax.experimental import pallas as pl
from jax.experimental.pallas import tpu as pltpu

N_RAYS = 32768
N_SURF = 16
NBOUNCES = 100
EPS = 1e-4
LANES = 128
ROWS = N_RAYS // LANES
BLK = 16
GRID = ROWS // BLK


def _body(ctr_ref, rad_ref, refl_ref,
          px_ref, py_ref, pz_ref, dx_ref, dy_ref, dz_ref, it_ref,
          opx_ref, opy_ref, opz_ref, odx_ref, ody_ref, odz_ref, oit_ref):
    px = px_ref[...] * 5.0
    py = py_ref[...] * 5.0
    pz = pz_ref[...] * 5.0
    dx = dx_ref[...]
    dy = dy_ref[...]
    dz = dz_ref[...]
    inten = it_ref[...]

    nrm = jnp.sqrt((dx * dx + dz * dz) + dy * dy) + 1e-12
    dx = dx / nrm
    dy = dy / nrm
    dz = dz / nrm

    inf = jnp.float32(jnp.inf)

    surf = []
    for s in range(N_SURF):
        surf.append((ctr_ref[s, 0] * 10.0, ctr_ref[s, 1] * 10.0,
                     ctr_ref[s, 2] * 10.0, rad_ref[s] * 4.0 + 1.0,
                     refl_ref[s]))

    def chain(state, lo, hi):
        px, py, pz, dx, dy, dz = state
        best_t = jnp.full(px.shape, inf, jnp.float32)
        bcx = jnp.zeros(px.shape, jnp.float32)
        bcy = jnp.zeros(px.shape, jnp.float32)
        bcz = jnp.zeros(px.shape, jnp.float32)
        brad = jnp.ones(px.shape, jnp.float32)
        brefl = jnp.zeros(px.shape, jnp.float32)
        for s in range(lo, hi):
            cx, cy, cz, r, rf = surf[s]
            ox = px - cx
            oy = py - cy
            oz = pz - cz
            b = (ox * dx + oy * dy) + oz * dz
            c = ((ox * ox + oy * oy) + oz * oz) - r * r
            disc = b * b - c
            valid = disc > 0.0
            sq = jnp.sqrt(disc)
            t0 = -b - sq
            t1 = -b + sq
            t = jnp.where(valid & (t0 > EPS), t0,
                          jnp.where(valid & (t1 > EPS), t1, inf))
            take = t < best_t
            best_t = jnp.where(take, t, best_t)
            bcx = jnp.where(take, cx, bcx)
            bcy = jnp.where(take, cy, bcy)
            bcz = jnp.where(take, cz, bcz)
            brad = jnp.where(take, r, brad)
            brefl = jnp.where(take, rf, brefl)
        return best_t, bcx, bcy, bcz, brad, brefl

    def bounce(_, state):
        px, py, pz, dx, dy, dz, inten = state
        rays = (px, py, pz, dx, dy, dz)
        half = N_SURF // 2
        ta, axc, ayc, azc, ar, af = chain(rays, 0, half)
        tb, bxc, byc, bzc, br, bf = chain(rays, half, N_SURF)
        tk = tb < ta
        best_t = jnp.where(tk, tb, ta)
        bcx = jnp.where(tk, bxc, axc)
        bcy = jnp.where(tk, byc, ayc)
        bcz = jnp.where(tk, bzc, azc)
        brad = jnp.where(tk, br, ar)
        brefl = jnp.where(tk, bf, af)
        hit = best_t < inf
        active = hit & (inten > 0.0)
        t_safe = jnp.where(hit, best_t, 0.0)
        hx = px + t_safe * dx
        hy = py + t_safe * dy
        hz = pz + t_safe * dz
        nx = (hx - bcx) / brad
        ny = (hy - bcy) / brad
        nz = (hz - bcz) / brad
        dn = (dx * nx + dz * nz) + dy * ny
        k2 = 2.0 * dn
        ndx = dx - k2 * nx
        ndy = dy - k2 * ny
        ndz = dz - k2 * nz
        ni = inten * brefl
        px = jnp.where(active, hx, px)
        py = jnp.where(active, hy, py)
        pz = jnp.where(active, hz, pz)
        dx = jnp.where(active, ndx, dx)
        dy = jnp.where(active, ndy, dy)
        dz = jnp.where(active, ndz, dz)
        inten = jnp.where(active, ni, inten)
        return px, py, pz, dx, dy, dz, inten

    UNROLL = 10

    def bounce4(i, state):
        for _ in range(UNROLL):
            state = bounce(i, state)
        return state

    px, py, pz, dx, dy, dz, inten = lax.fori_loop(
        0, NBOUNCES // UNROLL, bounce4, (px, py, pz, dx, dy, dz, inten))

    opx_ref[...] = px
    opy_ref[...] = py
    opz_ref[...] = pz
    odx_ref[...] = dx
    ody_ref[...] = dy
    odz_ref[...] = dz
    oit_ref[...] = inten


def kernel(pos, dir, intensity, centers, radii, reflectivity):
    comps = [pos[:, 0], pos[:, 1], pos[:, 2],
             dir[:, 0], dir[:, 1], dir[:, 2], intensity]
    planes = [c.reshape(ROWS, LANES) for c in comps]

    vspec = pl.BlockSpec((BLK, LANES), lambda i: (i, 0))
    sspec = pl.BlockSpec(memory_space=pltpu.MemorySpace.SMEM)
    out = pl.pallas_call(
        _body,
        grid=(GRID,),
        in_specs=[sspec, sspec, sspec] + [vspec] * 7,
        out_specs=[vspec] * 7,
        out_shape=[jax.ShapeDtypeStruct((ROWS, LANES), jnp.float32)] * 7,
    )(centers, radii, reflectivity, *planes)

    flat = [o.reshape(N_RAYS) for o in out]
    return jnp.stack(flat, axis=1)


# speedup vs baseline: 1.1055x; 1.0067x over previous
import jax
import jax.numpy as jnp
from jax import lax
from jax.experimental import pallas as pl
from jax.experimental.pallas import tpu as pltpu

N_RAYS = 32768
N_SURF = 16
NBOUNCES = 100
EPS = 1e-4
LANES = 128
ROWS = N_RAYS // LANES
BLK = 16
GRID = ROWS // BLK


def _body(ctr_ref, rad_ref, refl_ref,
          px_ref, py_ref, pz_ref, dx_ref, dy_ref, dz_ref, it_ref,
          opx_ref, opy_ref, opz_ref, odx_ref, ody_ref, odz_ref, oit_ref):
    px = px_ref[...] * 5.0
    py = py_ref[...] * 5.0
    pz = pz_ref[...] * 5.0
    dx = dx_ref[...]
    dy = dy_ref[...]
    dz = dz_ref[...]
    inten = it_ref[...]

    nrm = jnp.sqrt((dx * dx + dz * dz) + dy * dy) + 1e-12
    dx = dx / nrm
    dy = dy / nrm
    dz = dz / nrm

    inf = jnp.float32(jnp.inf)

    surf = []
    for s in range(N_SURF):
        surf.append((ctr_ref[s, 0] * 10.0, ctr_ref[s, 1] * 10.0,
                     ctr_ref[s, 2] * 10.0, rad_ref[s] * 4.0 + 1.0,
                     refl_ref[s]))

    def chain(state, lo, hi):
        px, py, pz, dx, dy, dz = state
        best_t = jnp.full(px.shape, inf, jnp.float32)
        bcx = jnp.zeros(px.shape, jnp.float32)
        bcy = jnp.zeros(px.shape, jnp.float32)
        bcz = jnp.zeros(px.shape, jnp.float32)
        brad = jnp.ones(px.shape, jnp.float32)
        brefl = jnp.zeros(px.shape, jnp.float32)
        for s in range(lo, hi):
            cx, cy, cz, r, rf = surf[s]
            ox = px - cx
            oy = py - cy
            oz = pz - cz
            b = (ox * dx + oy * dy) + oz * dz
            c = ((ox * ox + oy * oy) + oz * oz) - r * r
            disc = b * b - c
            valid = disc > 0.0
            sq = jnp.sqrt(disc)
            t0 = -b - sq
            t1 = -b + sq
            t = jnp.where(valid & (t0 > EPS), t0,
                          jnp.where(valid & (t1 > EPS), t1, inf))
            take = t < best_t
            best_t = jnp.where(take, t, best_t)
            bcx = jnp.where(take, cx, bcx)
            bcy = jnp.where(take, cy, bcy)
            bcz = jnp.where(take, cz, bcz)
            brad = jnp.where(take, r, brad)
            brefl = jnp.where(take, rf, brefl)
        return best_t, bcx, bcy, bcz, brad, brefl

    def bounce(_, state):
        px, py, pz, dx, dy, dz, inten = state
        rays = (px, py, pz, dx, dy, dz)
        half = N_SURF // 2
        ta, axc, ayc, azc, ar, af = chain(rays, 0, half)
        tb, bxc, byc, bzc, br, bf = chain(rays, half, N_SURF)
        tk = tb < ta
        best_t = jnp.where(tk, tb, ta)
        bcx = jnp.where(tk, bxc, axc)
        bcy = jnp.where(tk, byc, ayc)
        bcz = jnp.where(tk, bzc, azc)
        brad = jnp.where(tk, br, ar)
        brefl = jnp.where(tk, bf, af)
        hit = best_t < inf
        active = hit & (inten > 0.0)
        t_safe = jnp.where(hit, best_t, 0.0)
        hx = px + t_safe * dx
        hy = py + t_safe * dy
        hz = pz + t_safe * dz
        nx = (hx - bcx) / brad
        ny = (hy - bcy) / brad
        nz = (hz - bcz) / brad
        dn = (dx * nx + dz * nz) + dy * ny
        k2 = 2.0 * dn
        ndx = dx - k2 * nx
        ndy = dy - k2 * ny
        ndz = dz - k2 * nz
        ni = inten * brefl
        px = jnp.where(active, hx, px)
        py = jnp.where(active, hy, py)
        pz = jnp.where(active, hz, pz)
        dx = jnp.where(active, ndx, dx)
        dy = jnp.where(active, ndy, dy)
        dz = jnp.where(active, ndz, dz)
        inten = jnp.where(active, ni, inten)
        return px, py, pz, dx, dy, dz, inten

    UNROLL = 25

    def bounce4(i, state):
        for _ in range(UNROLL):
            state = bounce(i, state)
        return state

    px, py, pz, dx, dy, dz, inten = lax.fori_loop(
        0, NBOUNCES // UNROLL, bounce4, (px, py, pz, dx, dy, dz, inten))

    opx_ref[...] = px
    opy_ref[...] = py
    opz_ref[...] = pz
    odx_ref[...] = dx
    ody_ref[...] = dy
    odz_ref[...] = dz
    oit_ref[...] = inten


def kernel(pos, dir, intensity, centers, radii, reflectivity):
    comps = [pos[:, 0], pos[:, 1], pos[:, 2],
             dir[:, 0], dir[:, 1], dir[:, 2], intensity]
    planes = [c.reshape(ROWS, LANES) for c in comps]

    vspec = pl.BlockSpec((BLK, LANES), lambda i: (i, 0))
    sspec = pl.BlockSpec(memory_space=pltpu.MemorySpace.SMEM)
    out = pl.pallas_call(
        _body,
        grid=(GRID,),
        in_specs=[sspec, sspec, sspec] + [vspec] * 7,
        out_specs=[vspec] * 7,
        out_shape=[jax.ShapeDtypeStruct((ROWS, LANES), jnp.float32)] * 7,
    )(centers, radii, reflectivity, *planes)

    flat = [o.reshape(N_RAYS) for o in out]
    return jnp.stack(flat, axis=1)
